# baseline (speedup 1.0000x reference)
import jax
import jax.numpy as jnp
from jax.experimental import pallas as pl
from jax.experimental.pallas import tpu as pltpu
from jax.experimental.pallas import tpu_sc as plsc

EPS = 1e-12
GR = 64
ROWS = 512
CHUNKS = 5


def _sc_gather(table, idx):
    m = idx.shape[1]
    dim = table.shape[1]

    @pl.kernel(
        out_type=jax.ShapeDtypeStruct((m, dim), table.dtype),
        mesh=plsc.VectorSubcoreMesh(core_axis_name="core",
                                    subcore_axis_name="subcore"),
    )
    def k(tab_hbm, i_hbm, o_hbm):
        def body(indices, i_vmem, o_vmem):
            (step,) = indices
            base = (step % 2) * GR
            pltpu.sync_copy(tab_hbm.at[i_vmem.at[0, pl.ds(base, GR)]], o_vmem)

        pltpu.emit_pipeline(
            body,
            grid=(m // GR,),
            in_specs=[pl.BlockSpec((1, 2 * GR), index_map=lambda i: (0, i // 2))],
            out_specs=[pl.BlockSpec((GR, dim), index_map=lambda i: (i, 0))],
            core_axis_name=("core", "subcore"),
            dimension_semantics=(pltpu.PARALLEL,),
            _explicit_indices=True,
        )(i_hbm, o_hbm)

    return k(table, idx)


def _tc_layernorm_chunk(x, pos, gamma, beta, carry, b, sig, sig_chunk, dim,
                        chunk):
    nblocks = (sig_chunk * b) // ROWS
    per_band = b // ROWS
    s_off = chunk * sig_chunk

    def body(x_ref, p_ref, g_ref, bt_ref, ones_ref, *rest):
        o_ref = rest[-1]
        v = x_ref[...] + p_ref[0]
        vb = v.astype(jnp.bfloat16)
        s1 = jnp.dot(vb, ones_ref[...], preferred_element_type=jnp.float32)
        s2 = jnp.dot(vb * vb, ones_ref[...],
                     preferred_element_type=jnp.float32)
        mean = s1[:, :1] * (1.0 / dim)
        var = s2[:, :1] * (1.0 / dim) - mean * mean
        inv = jax.lax.rsqrt(var + EPS)
        y = (v - mean) * inv * g_ref[...] + bt_ref[...]
        o_ref[...] = y.reshape(1, ROWS, dim)

    in_specs = [
        pl.BlockSpec((ROWS, dim), lambda i: (i, 0)),
        pl.BlockSpec((1, 1, dim), lambda i: (s_off + i // per_band, 0, 0)),
        pl.BlockSpec((1, dim), lambda i: (0, 0)),
        pl.BlockSpec((1, dim), lambda i: (0, 0)),
        pl.BlockSpec((dim, 128), lambda i: (0, 0)),
    ]
    args = [x, pos, gamma, beta,
            jnp.ones((dim, 128), dtype=jnp.bfloat16)]
    aliases = {}
    if carry is not None:
        in_specs.append(pl.BlockSpec(memory_space=pltpu.MemorySpace.HBM))
        args.append(carry)
        aliases = {5: 0}

    return pl.pallas_call(
        body,
        grid=(nblocks,),
        in_specs=in_specs,
        out_specs=pl.BlockSpec(
            (1, ROWS, dim),
            lambda i: (s_off + i // per_band, i % per_band, 0)),
        out_shape=jax.ShapeDtypeStruct((sig, b, dim), jnp.float32),
        input_output_aliases=aliases,
    )(*args)


def kernel(news_batch, word_embeddings, pos_embedding, gamma, beta):
    b, sig = news_batch.shape
    vocab, dim = word_embeddings.shape
    pos2 = pos_embedding.reshape(sig, 1, dim)
    g2 = gamma.reshape(1, dim)
    b2 = beta.reshape(1, dim)

    idx_t = news_batch.T.astype(jnp.int32)
    sc = sig // CHUNKS
    gathers = []
    for c in range(CHUNKS):
        idx = idx_t[c * sc:(c + 1) * sc].reshape(1, sc * b)
        gathers.append(_sc_gather(word_embeddings, idx))

    out = None
    for c in range(CHUNKS):
        out = _tc_layernorm_chunk(gathers[c], pos2, g2, b2, out,
                                  b, sig, sc, dim, c)
    return out.transpose(1, 0, 2)


# speedup vs baseline: 1.0371x; 1.0371x over previous
import jax
import jax.numpy as jnp
from jax.experimental import pallas as pl
from jax.experimental.pallas import tpu as pltpu
from jax.experimental.pallas import tpu_sc as plsc

EPS = 1e-12
GR = 64
ROWS = 512
CHUNKS = 5


def _sc_gather(table, idx):
    m = idx.shape[1]
    dim = table.shape[1]

    @pl.kernel(
        out_type=jax.ShapeDtypeStruct((m, dim), table.dtype),
        mesh=plsc.VectorSubcoreMesh(core_axis_name="core",
                                    subcore_axis_name="subcore"),
    )
    def k(tab_hbm, i_hbm, o_hbm):
        def body(indices, i_vmem, o_vmem):
            (step,) = indices
            base = (step % 2) * GR
            pltpu.sync_copy(tab_hbm.at[i_vmem.at[0, pl.ds(base, GR)]], o_vmem)

        pltpu.emit_pipeline(
            body,
            grid=(m // GR,),
            in_specs=[pl.BlockSpec((1, 2 * GR), index_map=lambda i: (0, i // 2))],
            out_specs=[pl.BlockSpec((GR, dim), index_map=lambda i: (i, 0))],
            core_axis_name=("core", "subcore"),
            dimension_semantics=(pltpu.PARALLEL,),
            _explicit_indices=True,
        )(i_hbm, o_hbm)

    return k(table, idx)


def _tc_layernorm_chunk(x, pos, gamma, beta, carry, b, sig, sig_chunk, dim,
                        chunk):
    nblocks = (sig_chunk * b) // ROWS
    per_band = b // ROWS
    s_off = chunk * sig_chunk

    def body(x_ref, p_ref, g_ref, bt_ref, *rest):
        o_ref = rest[-1]
        v = x_ref[...] + p_ref[0]
        s1 = jnp.sum(v, axis=-1, keepdims=True)
        s2 = jnp.sum(v * v, axis=-1, keepdims=True)
        mean = s1 * (1.0 / dim)
        var = s2 * (1.0 / dim) - mean * mean
        inv = jax.lax.rsqrt(var + EPS)
        y = (v - mean) * inv * g_ref[...] + bt_ref[...]
        o_ref[...] = y.reshape(1, ROWS, dim)

    in_specs = [
        pl.BlockSpec((ROWS, dim), lambda i: (i, 0)),
        pl.BlockSpec((1, 1, dim), lambda i: (s_off + i // per_band, 0, 0)),
        pl.BlockSpec((1, dim), lambda i: (0, 0)),
        pl.BlockSpec((1, dim), lambda i: (0, 0)),
    ]
    args = [x, pos, gamma, beta]
    aliases = {}
    if carry is not None:
        in_specs.append(pl.BlockSpec(memory_space=pltpu.MemorySpace.HBM))
        args.append(carry)
        aliases = {4: 0}

    return pl.pallas_call(
        body,
        grid=(nblocks,),
        in_specs=in_specs,
        out_specs=pl.BlockSpec(
            (1, ROWS, dim),
            lambda i: (s_off + i // per_band, i % per_band, 0)),
        out_shape=jax.ShapeDtypeStruct((sig, b, dim), jnp.float32),
        input_output_aliases=aliases,
    )(*args)


def kernel(news_batch, word_embeddings, pos_embedding, gamma, beta):
    b, sig = news_batch.shape
    vocab, dim = word_embeddings.shape
    pos2 = pos_embedding.reshape(sig, 1, dim)
    g2 = gamma.reshape(1, dim)
    b2 = beta.reshape(1, dim)

    idx_t = news_batch.T.astype(jnp.int32)
    sc = sig // CHUNKS
    gathers = []
    for c in range(CHUNKS):
        idx = idx_t[c * sc:(c + 1) * sc].reshape(1, sc * b)
        gathers.append(_sc_gather(word_embeddings, idx))

    out = None
    for c in range(CHUNKS):
        out = _tc_layernorm_chunk(gathers[c], pos2, g2, b2, out,
                                  b, sig, sc, dim, c)
    return out.transpose(1, 0, 2)


# speedup vs baseline: 1.0742x; 1.0357x over previous
import jax
import jax.numpy as jnp
from jax.experimental import pallas as pl
from jax.experimental.pallas import tpu as pltpu
from jax.experimental.pallas import tpu_sc as plsc

EPS = 1e-12
GR = 64
ROWS = 1024
CHUNKS = 5


def _sc_gather(table, idx):
    m = idx.shape[1]
    dim = table.shape[1]

    @pl.kernel(
        out_type=jax.ShapeDtypeStruct((m, dim), table.dtype),
        mesh=plsc.VectorSubcoreMesh(core_axis_name="core",
                                    subcore_axis_name="subcore"),
    )
    def k(tab_hbm, i_hbm, o_hbm):
        def body(indices, i_vmem, o_vmem):
            (step,) = indices
            base = (step % 2) * GR
            pltpu.sync_copy(tab_hbm.at[i_vmem.at[0, pl.ds(base, GR)]], o_vmem)

        pltpu.emit_pipeline(
            body,
            grid=(m // GR,),
            in_specs=[pl.BlockSpec((1, 2 * GR), index_map=lambda i: (0, i // 2))],
            out_specs=[pl.BlockSpec((GR, dim), index_map=lambda i: (i, 0))],
            core_axis_name=("core", "subcore"),
            dimension_semantics=(pltpu.PARALLEL,),
            _explicit_indices=True,
        )(i_hbm, o_hbm)

    return k(table, idx)


def _tc_layernorm_chunk(x, pos, gamma, beta, carry, b, sig, sig_chunk, dim,
                        chunk):
    nblocks = (sig_chunk * b) // ROWS
    per_band = b // ROWS
    s_off = chunk * sig_chunk

    def body(x_ref, p_ref, g_ref, bt_ref, *rest):
        o_ref = rest[-1]
        v = x_ref[...] + p_ref[0]
        s1 = jnp.sum(v, axis=-1, keepdims=True)
        s2 = jnp.sum(v * v, axis=-1, keepdims=True)
        mean = s1 * (1.0 / dim)
        var = s2 * (1.0 / dim) - mean * mean
        inv = jax.lax.rsqrt(var + EPS)
        y = (v - mean) * inv * g_ref[...] + bt_ref[...]
        o_ref[...] = y.reshape(1, ROWS, dim)

    in_specs = [
        pl.BlockSpec((ROWS, dim), lambda i: (i, 0)),
        pl.BlockSpec((1, 1, dim), lambda i: (s_off + i // per_band, 0, 0)),
        pl.BlockSpec((1, dim), lambda i: (0, 0)),
        pl.BlockSpec((1, dim), lambda i: (0, 0)),
    ]
    args = [x, pos, gamma, beta]
    aliases = {}
    if carry is not None:
        in_specs.append(pl.BlockSpec(memory_space=pltpu.MemorySpace.HBM))
        args.append(carry)
        aliases = {4: 0}

    return pl.pallas_call(
        body,
        grid=(nblocks,),
        in_specs=in_specs,
        out_specs=pl.BlockSpec(
            (1, ROWS, dim),
            lambda i: (s_off + i // per_band, i % per_band, 0)),
        out_shape=jax.ShapeDtypeStruct((sig, b, dim), jnp.float32),
        input_output_aliases=aliases,
    )(*args)


def kernel(news_batch, word_embeddings, pos_embedding, gamma, beta):
    b, sig = news_batch.shape
    vocab, dim = word_embeddings.shape
    pos2 = pos_embedding.reshape(sig, 1, dim)
    g2 = gamma.reshape(1, dim)
    b2 = beta.reshape(1, dim)

    idx_t = news_batch.T.astype(jnp.int32)
    sc = sig // CHUNKS
    gathers = []
    for c in range(CHUNKS):
        idx = idx_t[c * sc:(c + 1) * sc].reshape(1, sc * b)
        gathers.append(_sc_gather(word_embeddings, idx))

    out = None
    for c in range(CHUNKS):
        out = _tc_layernorm_chunk(gathers[c], pos2, g2, b2, out,
                                  b, sig, sc, dim, c)
    return out.transpose(1, 0, 2)


# speedup vs baseline: 1.0797x; 1.0051x over previous
import jax
import jax.numpy as jnp
from jax.experimental import pallas as pl
from jax.experimental.pallas import tpu as pltpu
from jax.experimental.pallas import tpu_sc as plsc

EPS = 1e-12
GR = 64
ROWS = 1024
SIZES = (4, 12, 14, 14, 6)


def _sc_gather(table, idx):
    m = idx.shape[1]
    dim = table.shape[1]

    @pl.kernel(
        out_type=jax.ShapeDtypeStruct((m, dim), table.dtype),
        mesh=plsc.VectorSubcoreMesh(core_axis_name="core",
                                    subcore_axis_name="subcore"),
    )
    def k(tab_hbm, i_hbm, o_hbm):
        def body(indices, i_vmem, o_vmem):
            (step,) = indices
            base = (step % 2) * GR
            pltpu.sync_copy(tab_hbm.at[i_vmem.at[0, pl.ds(base, GR)]], o_vmem)

        pltpu.emit_pipeline(
            body,
            grid=(m // GR,),
            in_specs=[pl.BlockSpec((1, 2 * GR), index_map=lambda i: (0, i // 2))],
            out_specs=[pl.BlockSpec((GR, dim), index_map=lambda i: (i, 0))],
            core_axis_name=("core", "subcore"),
            dimension_semantics=(pltpu.PARALLEL,),
            _explicit_indices=True,
        )(i_hbm, o_hbm)

    return k(table, idx)


def _tc_layernorm_chunk(x, pos, gamma, beta, carry, b, sig, sig_chunk, dim,
                        s_off):
    nblocks = (sig_chunk * b) // ROWS
    per_band = b // ROWS

    def body(x_ref, p_ref, g_ref, bt_ref, *rest):
        o_ref = rest[-1]
        v = x_ref[...] + p_ref[0]
        s1 = jnp.sum(v, axis=-1, keepdims=True)
        s2 = jnp.sum(v * v, axis=-1, keepdims=True)
        mean = s1 * (1.0 / dim)
        var = s2 * (1.0 / dim) - mean * mean
        inv = jax.lax.rsqrt(var + EPS)
        y = (v - mean) * inv * g_ref[...] + bt_ref[...]
        o_ref[...] = y.reshape(1, ROWS, dim)

    in_specs = [
        pl.BlockSpec((ROWS, dim), lambda i: (i, 0)),
        pl.BlockSpec((1, 1, dim), lambda i: (s_off + i // per_band, 0, 0)),
        pl.BlockSpec((1, dim), lambda i: (0, 0)),
        pl.BlockSpec((1, dim), lambda i: (0, 0)),
    ]
    args = [x, pos, gamma, beta]
    aliases = {}
    if carry is not None:
        in_specs.append(pl.BlockSpec(memory_space=pltpu.MemorySpace.HBM))
        args.append(carry)
        aliases = {4: 0}

    return pl.pallas_call(
        body,
        grid=(nblocks,),
        in_specs=in_specs,
        out_specs=pl.BlockSpec(
            (1, ROWS, dim),
            lambda i: (s_off + i // per_band, i % per_band, 0)),
        out_shape=jax.ShapeDtypeStruct((sig, b, dim), jnp.float32),
        input_output_aliases=aliases,
    )(*args)


def kernel(news_batch, word_embeddings, pos_embedding, gamma, beta):
    b, sig = news_batch.shape
    vocab, dim = word_embeddings.shape
    pos2 = pos_embedding.reshape(sig, 1, dim)
    g2 = gamma.reshape(1, dim)
    b2 = beta.reshape(1, dim)

    idx_t = news_batch.T.astype(jnp.int32)
    offs = [sum(SIZES[:c]) for c in range(len(SIZES))]
    gathers = []
    for c, sz in enumerate(SIZES):
        idx = idx_t[offs[c]:offs[c] + sz].reshape(1, sz * b)
        gathers.append(_sc_gather(word_embeddings, idx))

    out = None
    for c, sz in enumerate(SIZES):
        out = _tc_layernorm_chunk(gathers[c], pos2, g2, b2, out,
                                  b, sig, sz, dim, offs[c])
    return out.transpose(1, 0, 2)
